# Initial kernel scaffold; baseline (speedup 1.0000x reference)
import jax
import jax.numpy as jnp
from jax.experimental import pallas as pl


def kernel(tensor, gamma, beta, rand1, rand2):
    raise NotImplementedError("write your pallas kernel here")


# speedup vs baseline: 1.7781x; 1.7781x over previous
import functools

import jax
import jax.numpy as jnp
from jax import lax
from jax.experimental import pallas as pl
from jax.experimental.pallas import tpu as pltpu
from jax.experimental.pallas import tpu_sc as plsc

B, C, H, W = 256, 3, 224, 224
PLANE = H * W
NBLK = W // 16
PLANES = B * C
NWORKERS = 32
PER_W = PLANES // NWORKERS


def _sc_transform(flat_in, scale, off, flip):
    mesh = plsc.VectorSubcoreMesh(core_axis_name="c", subcore_axis_name="s")

    @functools.partial(
        pl.kernel,
        out_type=jax.ShapeDtypeStruct((PLANES * PLANE,), jnp.float32),
        mesh=mesh,
        scratch_types=[
            pltpu.VMEM((PLANE,), jnp.float32),
            pltpu.VMEM((PLANE,), jnp.float32),
            pltpu.VMEM((PLANES + 16,), jnp.float32),
            pltpu.VMEM((PLANES + 16,), jnp.float32),
            pltpu.VMEM((B + 16,), jnp.float32),
        ],
    )
    def k(in_hbm, scale_hbm, off_hbm, flip_hbm, out_hbm,
          in_v, out_v, scale_v, off_v, flip_v):
        wid = lax.axis_index("c") * 16 + lax.axis_index("s")
        pltpu.sync_copy(scale_hbm, scale_v.at[pl.ds(0, PLANES)])
        pltpu.sync_copy(off_hbm, off_v.at[pl.ds(0, PLANES)])
        pltpu.sync_copy(flip_hbm, flip_v.at[pl.ds(0, B)])

        def plane_body(i, carry):
            p = wid * PER_W + i
            s = lax.div(p, C)
            base = p * PLANE
            pltpu.sync_copy(in_hbm.at[pl.ds(base, PLANE)], in_v)
            sv = jnp.full((16,), scale_v[pl.ds(p, 16)][0])
            ov = jnp.full((16,), off_v[pl.ds(p, 16)][0])
            fv = jnp.full((16,), flip_v[pl.ds(s, 16)][0])

            def row_body(h, rcarry):
                ro = h * W
                xs = [in_v[pl.ds(ro + 16 * j, 16)] for j in range(NBLK)]
                for j in range(NBLK):
                    rev = xs[NBLK - 1 - j][::-1]
                    v = (xs[j] + fv * (rev - xs[j])) * sv + ov
                    out_v[pl.ds(ro + 16 * j, 16)] = v
                return rcarry

            lax.fori_loop(0, H, row_body, 0)
            pltpu.sync_copy(out_v, out_hbm.at[pl.ds(base, PLANE)])
            return carry

        lax.fori_loop(0, PER_W, plane_body, 0)

    return k(flat_in, scale, off, flip)


def kernel(tensor, gamma, beta, rand1, rand2):
    mask1 = rand1 <= 0.5
    mask2 = rand2 <= 0.3
    scale = jnp.where(mask1[:, None], gamma[None, :], 1.0).reshape(-1)
    off = jnp.where(mask1[:, None], beta[None, :], 0.0).reshape(-1)
    flip = mask2.astype(jnp.float32)
    out = _sc_transform(tensor.reshape(-1), scale, off, flip)
    return out.reshape(tensor.shape)


# speedup vs baseline: 2.0311x; 1.1423x over previous
import functools

import jax
import jax.numpy as jnp
from jax import lax
from jax.experimental import pallas as pl
from jax.experimental.pallas import tpu as pltpu
from jax.experimental.pallas import tpu_sc as plsc

B, C, H, W = 256, 3, 224, 224
PLANE = H * W
HALF = PLANE // 2
HROWS = H // 2
NBLK = W // 16
PLANES = B * C
NWORKERS = 32
PER_W = PLANES // NWORKERS
NCHUNK = 2 * PER_W


def _sc_transform(flat_in, scale, off, flip):
    mesh = plsc.VectorSubcoreMesh(core_axis_name="c", subcore_axis_name="s")

    @functools.partial(
        pl.kernel,
        out_type=jax.ShapeDtypeStruct((PLANES * PLANE,), jnp.float32),
        mesh=mesh,
        scratch_types=[
            pltpu.VMEM((HALF,), jnp.float32),
            pltpu.VMEM((HALF,), jnp.float32),
            pltpu.VMEM((HALF,), jnp.float32),
            pltpu.VMEM((HALF,), jnp.float32),
            pltpu.VMEM((PLANES + 16,), jnp.float32),
            pltpu.VMEM((PLANES + 16,), jnp.float32),
            pltpu.VMEM((B + 16,), jnp.float32),
            pltpu.SemaphoreType.DMA,
            pltpu.SemaphoreType.DMA,
            pltpu.SemaphoreType.DMA,
            pltpu.SemaphoreType.DMA,
        ],
    )
    def k(in_hbm, scale_hbm, off_hbm, flip_hbm, out_hbm,
          in0, in1, out0, out1, scale_v, off_v, flip_v,
          isem0, isem1, osem0, osem1):
        ins = (in0, in1)
        outs = (out0, out1)
        isems = (isem0, isem1)
        osems = (osem0, osem1)
        wid = lax.axis_index("c") * 16 + lax.axis_index("s")
        pltpu.sync_copy(scale_hbm, scale_v.at[pl.ds(0, PLANES)])
        pltpu.sync_copy(off_hbm, off_v.at[pl.ds(0, PLANES)])
        pltpu.sync_copy(flip_hbm, flip_v.at[pl.ds(0, B)])

        def chunk_base(c):
            p = wid * PER_W + lax.div(c, 2)
            return p * PLANE + lax.rem(c, 2) * HALF

        def issue_in(c, b):
            pltpu.make_async_copy(
                in_hbm.at[pl.ds(chunk_base(c), HALF)], ins[b], isems[b]
            ).start()

        issue_in(0, 0)

        def group_body(g, carry):
            for b in range(2):
                c = 2 * g + b

                @pl.when(c + 1 < NCHUNK)
                def _():
                    issue_in(c + 1, b ^ 1)

                pltpu.make_async_copy(
                    in_hbm.at[pl.ds(chunk_base(c), HALF)], ins[b], isems[b]
                ).wait()

                @pl.when(c >= 2)
                def _():
                    pltpu.make_async_copy(
                        outs[b],
                        out_hbm.at[pl.ds(chunk_base(c - 2), HALF)],
                        osems[b],
                    ).wait()

                p = wid * PER_W + lax.div(c, 2)
                s = lax.div(p, C)
                sv = jnp.full((16,), scale_v[pl.ds(p, 16)][0])
                ov = jnp.full((16,), off_v[pl.ds(p, 16)][0])
                fv = jnp.full((16,), flip_v[pl.ds(s, 16)][0])
                in_b, out_b = ins[b], outs[b]

                def row_body(h, rcarry):
                    ro = h * W
                    xs = [in_b[pl.ds(ro + 16 * j, 16)] for j in range(NBLK)]
                    for j in range(NBLK):
                        rev = xs[NBLK - 1 - j][::-1]
                        v = (xs[j] + fv * (rev - xs[j])) * sv + ov
                        out_b[pl.ds(ro + 16 * j, 16)] = v
                    return rcarry

                lax.fori_loop(0, HROWS, row_body, 0)

                pltpu.make_async_copy(
                    out_b, out_hbm.at[pl.ds(chunk_base(c), HALF)], osems[b]
                ).start()
            return carry

        lax.fori_loop(0, PER_W, group_body, 0)

        for b in range(2):
            c = NCHUNK - 2 + b
            pltpu.make_async_copy(
                outs[b], out_hbm.at[pl.ds(chunk_base(c), HALF)], osems[b]
            ).wait()

    return k(flat_in, scale, off, flip)


def kernel(tensor, gamma, beta, rand1, rand2):
    mask1 = rand1 <= 0.5
    mask2 = rand2 <= 0.3
    scale = jnp.where(mask1[:, None], gamma[None, :], 1.0).reshape(-1)
    off = jnp.where(mask1[:, None], beta[None, :], 0.0).reshape(-1)
    flip = mask2.astype(jnp.float32)
    out = _sc_transform(tensor.reshape(-1), scale, off, flip)
    return out.reshape(tensor.shape)


# speedup vs baseline: 3.5302x; 1.7381x over previous
import functools

import jax
import jax.numpy as jnp
from jax import lax
from jax.experimental import pallas as pl
from jax.experimental.pallas import tpu as pltpu
from jax.experimental.pallas import tpu_sc as plsc

B, C, H, W = 256, 3, 224, 224
PLANE = H * W
HROWS = H // 2
HALF = HROWS * W
NBLK = W // 16
PLANES = B * C
NWORKERS = 32
PER_W = PLANES // NWORKERS
NCHUNK = 2 * PER_W


def _sc_transform(tensor, scale, off, flip):
    mesh = plsc.VectorSubcoreMesh(core_axis_name="c", subcore_axis_name="s")

    @functools.partial(
        pl.kernel,
        out_type=jax.ShapeDtypeStruct((B, C, H, W), jnp.float32),
        mesh=mesh,
        scratch_types=[
            pltpu.VMEM((HROWS, W), jnp.float32),
            pltpu.VMEM((HROWS, W), jnp.float32),
            pltpu.VMEM((HROWS, W), jnp.float32),
            pltpu.VMEM((HROWS, W), jnp.float32),
            pltpu.VMEM((PLANES + 16,), jnp.float32),
            pltpu.VMEM((PLANES + 16,), jnp.float32),
            pltpu.VMEM((B + 16,), jnp.float32),
            pltpu.SemaphoreType.DMA,
            pltpu.SemaphoreType.DMA,
            pltpu.SemaphoreType.DMA,
            pltpu.SemaphoreType.DMA,
        ],
    )
    def k(in_hbm, scale_hbm, off_hbm, flip_hbm, out_hbm,
          in0, in1, out0, out1, scale_v, off_v, flip_v,
          isem0, isem1, osem0, osem1):
        ins = (in0, in1)
        outs = (out0, out1)
        isems = (isem0, isem1)
        osems = (osem0, osem1)
        wid = lax.axis_index("c") * 16 + lax.axis_index("s")
        pltpu.sync_copy(scale_hbm, scale_v.at[pl.ds(0, PLANES)])
        pltpu.sync_copy(off_hbm, off_v.at[pl.ds(0, PLANES)])
        pltpu.sync_copy(flip_hbm, flip_v.at[pl.ds(0, B)])

        def chunk_slice(c):
            p = wid * PER_W + lax.div(c, 2)
            s = lax.div(p, C)
            ch = lax.rem(p, C)
            h0 = lax.rem(c, 2) * HROWS
            return s, ch, h0

        def issue_in(c, b):
            s, ch, h0 = chunk_slice(c)
            pltpu.make_async_copy(
                in_hbm.at[s, ch, pl.ds(h0, HROWS)], ins[b], isems[b]
            ).start()

        issue_in(0, 0)

        def group_body(g, carry):
            for b in range(2):
                c = 2 * g + b

                @pl.when(c + 1 < NCHUNK)
                def _():
                    issue_in(c + 1, b ^ 1)

                s, ch, h0 = chunk_slice(c)
                pltpu.make_async_copy(
                    in_hbm.at[s, ch, pl.ds(h0, HROWS)], ins[b], isems[b]
                ).wait()

                @pl.when(c >= 2)
                def _():
                    s2, ch2, h2 = chunk_slice(c - 2)
                    pltpu.make_async_copy(
                        outs[b],
                        out_hbm.at[s2, ch2, pl.ds(h2, HROWS)],
                        osems[b],
                    ).wait()

                p = wid * PER_W + lax.div(c, 2)
                sv = jnp.full((16,), scale_v[pl.ds(p, 16)][0])
                ov = jnp.full((16,), off_v[pl.ds(p, 16)][0])
                do_flip = flip_v[pl.ds(s, 16)][0] > 0.5
                in_b, out_b = ins[b], outs[b]

                def flip_rows():
                    def row_body(h, rcarry):
                        xs = [in_b[h, pl.ds(16 * j, 16)]
                              for j in range(NBLK)]
                        for j in range(NBLK):
                            rev = xs[NBLK - 1 - j][::-1]
                            out_b[h, pl.ds(16 * j, 16)] = rev * sv + ov
                        return rcarry

                    lax.fori_loop(0, HROWS, row_body, 0)

                def straight_rows():
                    def row_body(h, rcarry):
                        for j in range(NBLK):
                            x = in_b[h, pl.ds(16 * j, 16)]
                            out_b[h, pl.ds(16 * j, 16)] = x * sv + ov
                        return rcarry

                    lax.fori_loop(0, HROWS, row_body, 0)

                lax.cond(do_flip, flip_rows, straight_rows)

                pltpu.make_async_copy(
                    out_b, out_hbm.at[s, ch, pl.ds(h0, HROWS)], osems[b]
                ).start()
            return carry

        lax.fori_loop(0, PER_W, group_body, 0)

        for b in range(2):
            c = NCHUNK - 2 + b
            s, ch, h0 = chunk_slice(c)
            pltpu.make_async_copy(
                outs[b], out_hbm.at[s, ch, pl.ds(h0, HROWS)], osems[b]
            ).wait()

    return k(tensor, scale, off, flip)


def kernel(tensor, gamma, beta, rand1, rand2):
    mask1 = rand1 <= 0.5
    mask2 = rand2 <= 0.3
    scale = jnp.where(mask1[:, None], gamma[None, :], 1.0).reshape(-1)
    off = jnp.where(mask1[:, None], beta[None, :], 0.0).reshape(-1)
    flip = mask2.astype(jnp.float32)
    return _sc_transform(tensor, scale, off, flip)


# speedup vs baseline: 11.9274x; 3.3786x over previous
import functools

import jax
import jax.numpy as jnp
from jax import lax
from jax.experimental import pallas as pl
from jax.experimental.pallas import tpu as pltpu
from jax.experimental.pallas import tpu_sc as plsc

B, C, H, W = 256, 3, 224, 224
HB = B // 2
ROWS = C * H
NWORKERS = 32
PER_W = ROWS // NWORKERS
NCH = 2 * PER_W
NSB = HB // 16
WPAIRS = W // 2
NBUF = 4


def _sc_transform(tchw, p, q, ov):
    mesh = plsc.VectorSubcoreMesh(core_axis_name="c", subcore_axis_name="s")

    @functools.partial(
        pl.kernel,
        out_type=jax.ShapeDtypeStruct((C, H, W, B), jnp.float32),
        mesh=mesh,
        scratch_types=[
            pltpu.VMEM((W, HB), jnp.float32),
            pltpu.VMEM((W, HB), jnp.float32),
            pltpu.VMEM((W, HB), jnp.float32),
            pltpu.VMEM((W, HB), jnp.float32),
            pltpu.VMEM((C * B,), jnp.float32),
            pltpu.VMEM((C * B,), jnp.float32),
            pltpu.VMEM((C * B,), jnp.float32),
            pltpu.SemaphoreType.DMA,
            pltpu.SemaphoreType.DMA,
            pltpu.SemaphoreType.DMA,
            pltpu.SemaphoreType.DMA,
            pltpu.SemaphoreType.DMA,
            pltpu.SemaphoreType.DMA,
            pltpu.SemaphoreType.DMA,
            pltpu.SemaphoreType.DMA,
        ],
    )
    def k(in_hbm, p_hbm, q_hbm, ov_hbm, out_hbm,
          buf0, buf1, buf2, buf3, pv, qv, ovv,
          is0, is1, is2, is3, os0, os1, os2, os3):
        bufs = (buf0, buf1, buf2, buf3)
        isems = (is0, is1, is2, is3)
        osems = (os0, os1, os2, os3)
        wid = lax.axis_index("c") * 16 + lax.axis_index("s")
        pltpu.sync_copy(p_hbm, pv)
        pltpu.sync_copy(q_hbm, qv)
        pltpu.sync_copy(ov_hbm, ovv)

        def chunk_loc(j):
            r = wid * PER_W + lax.div(j, 2)
            sh = lax.rem(j, 2) * HB
            return lax.div(r, H), lax.rem(r, H), sh

        def issue_in(j, b):
            c, h, sh = chunk_loc(j)
            pltpu.make_async_copy(
                in_hbm.at[c, h, :, pl.ds(sh, HB)], bufs[b], isems[b]
            ).start()

        def wait_in(j, b):
            c, h, sh = chunk_loc(j)
            pltpu.make_async_copy(
                in_hbm.at[c, h, :, pl.ds(sh, HB)], bufs[b], isems[b]
            ).wait()

        def issue_out(j, b):
            c, h, sh = chunk_loc(j)
            pltpu.make_async_copy(
                bufs[b], out_hbm.at[c, h, :, pl.ds(sh, HB)], osems[b]
            ).start()

        def wait_out(j, b):
            c, h, sh = chunk_loc(j)
            pltpu.make_async_copy(
                bufs[b], out_hbm.at[c, h, :, pl.ds(sh, HB)], osems[b]
            ).wait()

        def compute(j, b):
            c, _, sh = chunk_loc(j)
            buf = bufs[b]
            for kk in range(NSB):
                base = c * B + sh + 16 * kk
                pk = pv[pl.ds(base, 16)]
                qk = qv[pl.ds(base, 16)]
                ok = ovv[pl.ds(base, 16)]
                sb = 16 * kk

                @plsc.parallel_loop(0, WPAIRS, unroll=4)
                def _(w, sb=sb, pk=pk, qk=qk, ok=ok):
                    m = W - 1 - w
                    x = buf[w, pl.ds(sb, 16)]
                    y = buf[m, pl.ds(sb, 16)]
                    buf[w, pl.ds(sb, 16)] = x * pk + y * qk + ok
                    buf[m, pl.ds(sb, 16)] = y * pk + x * qk + ok

        issue_in(0, 0)
        issue_in(1, 1)

        def group_body(g, carry):
            for b in range(NBUF):
                j = NBUF * g + b
                wait_in(j, b)
                bn = (b + 2) % NBUF

                @pl.when(j >= 2)
                def _():
                    wait_out(j - 2, bn)

                @pl.when(j + 2 < NCH)
                def _():
                    issue_in(j + 2, bn)

                compute(j, b)
                issue_out(j, b)
            return carry

        lax.fori_loop(0, NCH // NBUF, group_body, 0)

        for t in range(2):
            j = NCH - 2 + t
            wait_in(j, t)
            wait_out(j - 2, (t + 2) % NBUF)
            compute(j, t)
            issue_out(j, t)
        for t in range(2):
            wait_out(NCH - 2 + t, t)

    return k(tchw, p, q, ov)


def kernel(tensor, gamma, beta, rand1, rand2):
    mask1 = rand1 <= 0.5
    f = (rand2 <= 0.3).astype(jnp.float32)
    sc = jnp.where(mask1[None, :], gamma[:, None], 1.0)
    ovm = jnp.where(mask1[None, :], beta[:, None], 0.0)
    p = (sc * (1.0 - f)[None, :]).reshape(-1)
    q = (sc * f[None, :]).reshape(-1)
    ov = ovm.reshape(-1)
    tchw = jnp.transpose(tensor, (1, 2, 3, 0))
    out = _sc_transform(tchw, p, q, ov)
    return jnp.transpose(out, (3, 0, 1, 2))


# speedup vs baseline: 12.0090x; 1.0068x over previous
import functools

import jax
import jax.numpy as jnp
from jax import lax
from jax.experimental import pallas as pl
from jax.experimental.pallas import tpu as pltpu
from jax.experimental.pallas import tpu_sc as plsc

B, C, H, W = 256, 3, 224, 224
HB = B // 2
ROWS = C * H
NWORKERS = 32
PER_W = ROWS // NWORKERS
NCH = 2 * PER_W
NSB = HB // 16
WPAIRS = W // 2
NBUF = 4


def _sc_transform(tchw, p, q, ov):
    mesh = plsc.VectorSubcoreMesh(core_axis_name="c", subcore_axis_name="s")

    @functools.partial(
        pl.kernel,
        out_type=jax.ShapeDtypeStruct((C, H, W, B), jnp.float32),
        mesh=mesh,
        scratch_types=[
            pltpu.VMEM((W, HB), jnp.float32),
            pltpu.VMEM((W, HB), jnp.float32),
            pltpu.VMEM((W, HB), jnp.float32),
            pltpu.VMEM((W, HB), jnp.float32),
            pltpu.VMEM((C * B,), jnp.float32),
            pltpu.VMEM((C * B,), jnp.float32),
            pltpu.VMEM((C * B,), jnp.float32),
            pltpu.SemaphoreType.DMA,
            pltpu.SemaphoreType.DMA,
            pltpu.SemaphoreType.DMA,
            pltpu.SemaphoreType.DMA,
            pltpu.SemaphoreType.DMA,
            pltpu.SemaphoreType.DMA,
            pltpu.SemaphoreType.DMA,
            pltpu.SemaphoreType.DMA,
        ],
    )
    def k(in_hbm, p_hbm, q_hbm, ov_hbm, out_hbm,
          buf0, buf1, buf2, buf3, pv, qv, ovv,
          is0, is1, is2, is3, os0, os1, os2, os3):
        bufs = (buf0, buf1, buf2, buf3)
        isems = (is0, is1, is2, is3)
        osems = (os0, os1, os2, os3)
        wid = lax.axis_index("c") * 16 + lax.axis_index("s")
        pltpu.sync_copy(p_hbm, pv)
        pltpu.sync_copy(q_hbm, qv)
        pltpu.sync_copy(ov_hbm, ovv)

        def chunk_loc(j):
            r = lax.div(j, 2) * NWORKERS + wid
            sh = lax.rem(j, 2) * HB
            return lax.div(r, H), lax.rem(r, H), sh

        def issue_in(j, b):
            c, h, sh = chunk_loc(j)
            pltpu.make_async_copy(
                in_hbm.at[c, h, :, pl.ds(sh, HB)], bufs[b], isems[b]
            ).start()

        def wait_in(j, b):
            c, h, sh = chunk_loc(j)
            pltpu.make_async_copy(
                in_hbm.at[c, h, :, pl.ds(sh, HB)], bufs[b], isems[b]
            ).wait()

        def issue_out(j, b):
            c, h, sh = chunk_loc(j)
            pltpu.make_async_copy(
                bufs[b], out_hbm.at[c, h, :, pl.ds(sh, HB)], osems[b]
            ).start()

        def wait_out(j, b):
            c, h, sh = chunk_loc(j)
            pltpu.make_async_copy(
                bufs[b], out_hbm.at[c, h, :, pl.ds(sh, HB)], osems[b]
            ).wait()

        def compute(j, b):
            c, _, sh = chunk_loc(j)
            buf = bufs[b]
            for kk in range(NSB):
                base = c * B + sh + 16 * kk
                pk = pv[pl.ds(base, 16)]
                qk = qv[pl.ds(base, 16)]
                ok = ovv[pl.ds(base, 16)]
                sb = 16 * kk

                @plsc.parallel_loop(0, WPAIRS, unroll=4)
                def _(w, sb=sb, pk=pk, qk=qk, ok=ok):
                    m = W - 1 - w
                    x = buf[w, pl.ds(sb, 16)]
                    y = buf[m, pl.ds(sb, 16)]
                    buf[w, pl.ds(sb, 16)] = x * pk + y * qk + ok
                    buf[m, pl.ds(sb, 16)] = y * pk + x * qk + ok

        issue_in(0, 0)
        issue_in(1, 1)

        def group_body(g, carry):
            for b in range(NBUF):
                j = NBUF * g + b
                wait_in(j, b)
                bn = (b + 2) % NBUF

                @pl.when(j >= 2)
                def _():
                    wait_out(j - 2, bn)

                @pl.when(j + 2 < NCH)
                def _():
                    issue_in(j + 2, bn)

                compute(j, b)
                issue_out(j, b)
            return carry

        lax.fori_loop(0, NCH // NBUF, group_body, 0)

        for t in range(2):
            j = NCH - 2 + t
            wait_in(j, t)
            wait_out(j - 2, (t + 2) % NBUF)
            compute(j, t)
            issue_out(j, t)
        for t in range(2):
            wait_out(NCH - 2 + t, t)

    return k(tchw, p, q, ov)


def kernel(tensor, gamma, beta, rand1, rand2):
    mask1 = rand1 <= 0.5
    f = (rand2 <= 0.3).astype(jnp.float32)
    sc = jnp.where(mask1[None, :], gamma[:, None], 1.0)
    ovm = jnp.where(mask1[None, :], beta[:, None], 0.0)
    p = (sc * (1.0 - f)[None, :]).reshape(-1)
    q = (sc * f[None, :]).reshape(-1)
    ov = ovm.reshape(-1)
    tchw = jnp.transpose(tensor, (1, 2, 3, 0))
    out = _sc_transform(tchw, p, q, ov)
    return jnp.transpose(out, (3, 0, 1, 2))
